# baseline (speedup 1.0000x reference)
import functools

import jax
import jax.numpy as jnp
from jax import lax
from jax.experimental import pallas as pl
from jax.experimental.pallas import tpu as pltpu
from jax.experimental.pallas import tpu_sc as plsc

N_NODES = 10000
N_PAD = 10240
DUMP_ROW = 10008
E_EDGES = 160000
EL_PAIRS = 20000
D_IN = 256
D_H = 256
D_OUT = 128
W = 128

EC = 128
EB = 16
E_PAD = 163840
DC = 5
EL_PAD = 32 * DC * EC
ZR = 40

_MESH = plsc.VectorSubcoreMesh(core_axis_name="c", subcore_axis_name="s")


def _make_seg_sum(nc):
    def body(y_hbm, idx_hbm, out0_hbm, out1_hbm,
             stage0, stage1, dstv0, dstv1, rows0, rows1, zbuf,
             acc, gsem0, gsem1, ssem0, ssem1):
        c = lax.axis_index("c")
        s = lax.axis_index("s")
        base = (c * 16 + s) * nc

        def zrow(r, _):
            for k in range(W // 16):
                zbuf[r, pl.ds(k * 16, 16)] = jnp.zeros((16,), jnp.float32)
            return 0
        lax.fori_loop(0, ZR, zrow, 0)
        for k in range(640 // ZR):
            pltpu.sync_copy(zbuf, acc.at[pl.ds(s * 640 + k * ZR, ZR)])
        plsc.subcore_barrier()

        def load_dstv(stage, dstv):
            for k in range(W // 16):
                sl = pl.ds(k * 16, 16)
                dstv[sl] = stage[1, sl]

        pltpu.sync_copy(idx_hbm.at[base], stage0)
        load_dstv(stage0, dstv0)
        pltpu.async_copy(y_hbm.at[stage0.at[0]], rows0, gsem0)
        pltpu.sync_copy(idx_hbm.at[base + 1], stage1)
        load_dstv(stage1, dstv1)
        pltpu.async_copy(y_hbm.at[stage1.at[0]], rows1, gsem1)

        def it(j2, _):
            nxt = base + 2 * j2 + 2
            pltpu.make_async_copy(y_hbm.at[stage0.at[0]], rows0, gsem0).wait()
            pltpu.async_copy(rows0, acc.at[dstv0], ssem0, add=True)
            pltpu.make_async_copy(y_hbm.at[stage1.at[0]], rows1, gsem1).wait()
            pltpu.async_copy(rows1, acc.at[dstv1], ssem1, add=True)
            pltpu.sync_copy(idx_hbm.at[nxt], stage0)
            pltpu.make_async_copy(rows0, acc.at[dstv0], ssem0).wait()
            load_dstv(stage0, dstv0)
            pltpu.async_copy(y_hbm.at[stage0.at[0]], rows0, gsem0)
            pltpu.sync_copy(idx_hbm.at[nxt + 1], stage1)
            pltpu.make_async_copy(rows1, acc.at[dstv1], ssem1).wait()
            load_dstv(stage1, dstv1)
            pltpu.async_copy(y_hbm.at[stage1.at[0]], rows1, gsem1)
            return 0
        lax.fori_loop(0, nc // 2 - 1, it, 0)

        pltpu.make_async_copy(y_hbm.at[stage0.at[0]], rows0, gsem0).wait()
        pltpu.async_copy(rows0, acc.at[dstv0], ssem0, add=True)
        pltpu.make_async_copy(y_hbm.at[stage1.at[0]], rows1, gsem1).wait()
        pltpu.async_copy(rows1, acc.at[dstv1], ssem1, add=True)
        pltpu.make_async_copy(rows0, acc.at[dstv0], ssem0).wait()
        pltpu.make_async_copy(rows1, acc.at[dstv1], ssem1).wait()
        plsc.subcore_barrier()

        def copy_out(out_hbm):
            def _():
                for k in range(640 // ZR):
                    r0 = s * 640 + k * ZR
                    pltpu.sync_copy(acc.at[pl.ds(r0, ZR)], zbuf)
                    pltpu.sync_copy(zbuf, out_hbm.at[pl.ds(r0, ZR)])
            return _
        pl.when(c == 0)(copy_out(out0_hbm))
        pl.when(c == 1)(copy_out(out1_hbm))

    out = jax.ShapeDtypeStruct((N_PAD, W), jnp.float32)
    scratch = [
        pltpu.VMEM((2, EC), jnp.int32),
        pltpu.VMEM((2, EC), jnp.int32),
        pltpu.VMEM((EC,), jnp.int32),
        pltpu.VMEM((EC,), jnp.int32),
        pltpu.VMEM((EC, W), jnp.float32),
        pltpu.VMEM((EC, W), jnp.float32),
        pltpu.VMEM((ZR, W), jnp.float32),
        pltpu.VMEM_SHARED((N_PAD, W), jnp.float32),
        pltpu.SemaphoreType.DMA,
        pltpu.SemaphoreType.DMA,
        pltpu.SemaphoreType.DMA,
        pltpu.SemaphoreType.DMA,
    ]
    return functools.partial(pl.kernel, mesh=_MESH, out_type=[out, out],
                             scratch_types=scratch)(body)


def _cnt_body(dst_hbm, cnt0_hbm, cnt1_hbm, dstv, onesb, cbuf, cacc):
    c = lax.axis_index("c")
    s = lax.axis_index("s")
    w = c * 16 + s
    nc = 40

    def orow(r, _):
        for k in range(W // 16):
            onesb[r, pl.ds(k * 16, 16)] = jnp.ones((16,), jnp.float32)
        return 0
    lax.fori_loop(0, EC, orow, 0)

    def crow(r, _):
        for k in range(W // 16):
            cbuf[r, pl.ds(k * 16, 16)] = jnp.zeros((16,), jnp.float32)
        return 0
    lax.fori_loop(0, ZR, crow, 0)
    for k in range(640 // ZR):
        pltpu.sync_copy(cbuf, cacc.at[pl.ds(s * 640 + k * ZR, ZR)])
    plsc.subcore_barrier()

    def chunk(j, _):
        pltpu.sync_copy(dst_hbm.at[w * nc + j], dstv)
        pltpu.sync_copy(onesb, cacc.at[dstv], add=True)
        return 0
    lax.fori_loop(0, nc, chunk, 0)
    plsc.subcore_barrier()

    def copy_out(cnt_hbm):
        def _():
            for k in range(640 // ZR):
                r0 = s * 640 + k * ZR
                pltpu.sync_copy(cacc.at[pl.ds(r0, ZR)], cbuf)
                pltpu.sync_copy(cbuf, cnt_hbm.at[pl.ds(r0, ZR)])
        return _
    pl.when(c == 0)(copy_out(cnt0_hbm))
    pl.when(c == 1)(copy_out(cnt1_hbm))


def _make_cnt():
    out = jax.ShapeDtypeStruct((N_PAD, W), jnp.float32)
    scratch = [
        pltpu.VMEM((EC,), jnp.int32),
        pltpu.VMEM((EC, W), jnp.float32),
        pltpu.VMEM((ZR, W), jnp.float32),
        pltpu.VMEM_SHARED((N_PAD, W), jnp.float32),
    ]
    return functools.partial(
        pl.kernel, mesh=_MESH,
        out_type=[out, out],
        scratch_types=scratch)(_cnt_body)


def _decode_body(z_hbm, sidx_hbm, didx_hbm, out_hbm, sv, dv, srows, drows, sem):
    c = lax.axis_index("c")
    s = lax.axis_index("s")
    w = c * 16 + s
    pltpu.sync_copy(sidx_hbm.at[w], sv)
    pltpu.sync_copy(didx_hbm.at[w], dv)

    def chunk(j, _):
        a = pltpu.async_copy(z_hbm.at[sv.at[j]], srows, sem)
        b = pltpu.async_copy(z_hbm.at[dv.at[j]], drows, sem)
        a.wait()
        b.wait()

        def prow(r, _):
            for k in range(D_OUT // 16):
                sl = pl.ds(k * 16, 16)
                srows[r, sl] = srows[r, sl] * drows[r, sl]
            return 0
        lax.fori_loop(0, EC, prow, 0)
        pltpu.sync_copy(srows, out_hbm.at[pl.ds((w * DC + j) * EC, EC)])
        return 0
    lax.fori_loop(0, DC, chunk, 0)


def _make_decode():
    scratch = [
        pltpu.VMEM((DC, EC), jnp.int32),
        pltpu.VMEM((DC, EC), jnp.int32),
        pltpu.VMEM((EC, D_OUT), jnp.float32),
        pltpu.VMEM((EC, D_OUT), jnp.float32),
        pltpu.SemaphoreType.DMA,
    ]
    return functools.partial(
        pl.kernel, mesh=_MESH,
        out_type=jax.ShapeDtypeStruct((EL_PAD, D_OUT), jnp.float32),
        scratch_types=scratch)(_decode_body)


def _rowsum(p):
    nb, rb = 10, EL_PAD // 10

    def body(p_ref, o_ref):
        o_ref[...] = jnp.sum(p_ref[...], axis=1, keepdims=True)

    return pl.pallas_call(
        body,
        grid=(nb,),
        in_specs=[pl.BlockSpec((rb, D_OUT), lambda r: (r, 0))],
        out_specs=pl.BlockSpec((rb, 1), lambda r: (r, 0)),
        out_shape=jax.ShapeDtypeStruct((EL_PAD, 1), jnp.float32),
    )(p)


_DN = (((1,), (1,)), ((), ()))


def _matmul_split(h, Wl):
    din = h.shape[1]
    nb, rb = 10, 1000

    def body(h_ref, w_ref, o_ref):
        o_ref[...] = lax.dot_general(h_ref[...], w_ref[...], _DN,
                                     preferred_element_type=jnp.float32)

    return pl.pallas_call(
        body,
        grid=(2, nb),
        in_specs=[pl.BlockSpec((rb, din), lambda c, r: (r, 0)),
                  pl.BlockSpec((W, din), lambda c, r: (c, 0))],
        out_specs=pl.BlockSpec((rb, W), lambda c, r: (c * nb + r, 0)),
        out_shape=jax.ShapeDtypeStruct((2 * N_NODES, W), jnp.float32),
    )(h, Wl)


def _matmul_plain(h, Wl):
    din = h.shape[1]
    dout = Wl.shape[0]
    nb, rb = 10, 1000

    def body(h_ref, w_ref, o_ref):
        o_ref[...] = lax.dot_general(h_ref[...], w_ref[...], _DN,
                                     preferred_element_type=jnp.float32)

    return pl.pallas_call(
        body,
        grid=(nb,),
        in_specs=[pl.BlockSpec((rb, din), lambda r: (r, 0)),
                  pl.BlockSpec((dout, din), lambda r: (0, 0))],
        out_specs=pl.BlockSpec((rb, dout), lambda r: (r, 0)),
        out_shape=jax.ShapeDtypeStruct((N_NODES, dout), jnp.float32),
    )(h, Wl)


def _combine(s_lo, s_hi, cnt_lo, cnt_hi, h, Wr, bl, relu, mode):
    dout = Wr.shape[0]
    din = h.shape[1]
    nb, rb = 10, 1000

    def body(slo, shi, clo, chi, h_ref, wr_ref, bl_ref, o_ref):
        if mode == "concat":
            sfull = jnp.concatenate([slo[...], shi[...]], axis=1)
        else:
            sfull = slo[...] + shi[...]
        cc = jnp.maximum(clo[...][:, 0:1] + chi[...][:, 0:1], 1.0)
        val = (sfull / cc +
               lax.dot_general(h_ref[...], wr_ref[...], _DN,
                               preferred_element_type=jnp.float32) +
               bl_ref[...])
        o_ref[...] = jnp.maximum(val, 0.0) if relu else val

    return pl.pallas_call(
        body,
        grid=(nb,),
        in_specs=[pl.BlockSpec((rb, W), lambda r: (r, 0)),
                  pl.BlockSpec((rb, W), lambda r: (r, 0)),
                  pl.BlockSpec((rb, W), lambda r: (r, 0)),
                  pl.BlockSpec((rb, W), lambda r: (r, 0)),
                  pl.BlockSpec((rb, din), lambda r: (r, 0)),
                  pl.BlockSpec((dout, din), lambda r: (0, 0)),
                  pl.BlockSpec((1, dout), lambda r: (0, 0))],
        out_specs=pl.BlockSpec((rb, dout), lambda r: (r, 0)),
        out_shape=jax.ShapeDtypeStruct((N_NODES, dout), jnp.float32),
    )(s_lo, s_hi, cnt_lo, cnt_hi, h, Wr, bl.reshape(1, dout))


def kernel(x, edge_index, edge_label_index,
           Wl1, bl1, Wr1, Wl2, bl2, Wr2, Wl3, bl3, Wr3):
    src = edge_index[0]
    dst = edge_index[1]
    epad = E_PAD - E_EDGES
    src_p = jnp.concatenate([src, jnp.zeros((epad,), jnp.int32)])
    dst_p = jnp.concatenate([dst, jnp.full((epad,), DUMP_ROW, jnp.int32)])

    src_cs = jnp.concatenate([src_p, src_p + N_NODES]).reshape(2560, EC)
    dst_cs = jnp.tile(dst_p, 2).reshape(2560, EC)
    idx_cs = jnp.stack([src_cs, dst_cs], axis=1)
    src_es = src_p.reshape(1280, EC)
    dst_es = dst_p.reshape(1280, EC)
    idx_es = jnp.stack([src_es, dst_es], axis=1)

    seg_cs = _make_seg_sum(80)
    seg_es = _make_seg_sum(40)
    cnt_k = _make_cnt()
    decode = _make_decode()

    cnt0, cnt1 = cnt_k(dst_es)

    y1 = _matmul_split(x, Wl1)
    s1a, s1b = seg_cs(y1, idx_cs)
    h1 = _combine(s1a, s1b, cnt0, cnt1, x, Wr1, bl1, True, "concat")

    y2 = _matmul_split(h1, Wl2)
    s2a, s2b = seg_cs(y2, idx_cs)
    h2 = _combine(s2a, s2b, cnt0, cnt1, h1, Wr2, bl2, True, "concat")

    y3 = _matmul_plain(h2, Wl3)
    s3a, s3b = seg_es(y3, idx_es)
    z = _combine(s3a, s3b, cnt0, cnt1, h2, Wr3, bl3, False, "add")

    lpad = EL_PAD - EL_PAIRS
    es = jnp.concatenate([edge_label_index[0],
                          jnp.zeros((lpad,), jnp.int32)]).reshape(32, DC, EC)
    ed = jnp.concatenate([edge_label_index[1],
                          jnp.zeros((lpad,), jnp.int32)]).reshape(32, DC, EC)
    prods = decode(z, es, ed)
    dots = _rowsum(prods)
    return dots.reshape(EL_PAD)[:EL_PAIRS]


# speedup vs baseline: 1.0325x; 1.0325x over previous
import functools

import jax
import jax.numpy as jnp
from jax import lax
from jax.experimental import pallas as pl
from jax.experimental.pallas import tpu as pltpu
from jax.experimental.pallas import tpu_sc as plsc

N_NODES = 10000
N_PAD = 10240
DUMP_ROW = 10008
E_EDGES = 160000
EL_PAIRS = 20000
D_IN = 256
D_H = 256
D_OUT = 128
W = 128

EC = 128
EB = 16
E_PAD = 163840
DC = 5
EL_PAD = 32 * DC * EC
ZR = 40

_MESH = plsc.VectorSubcoreMesh(core_axis_name="c", subcore_axis_name="s")


def _make_seg_sum(nc):
    def body(y_hbm, src_hbm, dst_hbm, out0_hbm, out1_hbm,
             srcv0, srcv1, dstv0, dstv1, rows0, rows1, zbuf,
             acc, gsem0, gsem1, ssem0, ssem1):
        c = lax.axis_index("c")
        s = lax.axis_index("s")
        base = (c * 16 + s) * nc

        def zrow(r, _):
            for k in range(W // 16):
                zbuf[r, pl.ds(k * 16, 16)] = jnp.zeros((16,), jnp.float32)
            return 0
        lax.fori_loop(0, ZR, zrow, 0)
        for k in range(640 // ZR):
            pltpu.sync_copy(zbuf, acc.at[pl.ds(s * 640 + k * ZR, ZR)])
        plsc.subcore_barrier()

        pltpu.sync_copy(src_hbm.at[base], srcv0)
        pltpu.sync_copy(dst_hbm.at[base], dstv0)
        pltpu.async_copy(y_hbm.at[srcv0], rows0, gsem0)
        pltpu.sync_copy(src_hbm.at[base + 1], srcv1)
        pltpu.sync_copy(dst_hbm.at[base + 1], dstv1)
        pltpu.async_copy(y_hbm.at[srcv1], rows1, gsem1)

        def it(j2, _):
            nxt = base + 2 * j2 + 2
            pltpu.make_async_copy(y_hbm.at[srcv0], rows0, gsem0).wait()
            pltpu.async_copy(rows0, acc.at[dstv0], ssem0, add=True)
            pltpu.make_async_copy(y_hbm.at[srcv1], rows1, gsem1).wait()
            pltpu.async_copy(rows1, acc.at[dstv1], ssem1, add=True)
            pltpu.make_async_copy(rows0, acc.at[dstv0], ssem0).wait()
            pltpu.sync_copy(src_hbm.at[nxt], srcv0)
            pltpu.sync_copy(dst_hbm.at[nxt], dstv0)
            pltpu.async_copy(y_hbm.at[srcv0], rows0, gsem0)
            pltpu.make_async_copy(rows1, acc.at[dstv1], ssem1).wait()
            pltpu.sync_copy(src_hbm.at[nxt + 1], srcv1)
            pltpu.sync_copy(dst_hbm.at[nxt + 1], dstv1)
            pltpu.async_copy(y_hbm.at[srcv1], rows1, gsem1)
            return 0
        lax.fori_loop(0, nc // 2 - 1, it, 0)

        pltpu.make_async_copy(y_hbm.at[srcv0], rows0, gsem0).wait()
        pltpu.async_copy(rows0, acc.at[dstv0], ssem0, add=True)
        pltpu.make_async_copy(y_hbm.at[srcv1], rows1, gsem1).wait()
        pltpu.async_copy(rows1, acc.at[dstv1], ssem1, add=True)
        pltpu.make_async_copy(rows0, acc.at[dstv0], ssem0).wait()
        pltpu.make_async_copy(rows1, acc.at[dstv1], ssem1).wait()
        plsc.subcore_barrier()

        def copy_out(out_hbm):
            def _():
                for k in range(640 // ZR):
                    r0 = s * 640 + k * ZR
                    pltpu.sync_copy(acc.at[pl.ds(r0, ZR)], zbuf)
                    pltpu.sync_copy(zbuf, out_hbm.at[pl.ds(r0, ZR)])
            return _
        pl.when(c == 0)(copy_out(out0_hbm))
        pl.when(c == 1)(copy_out(out1_hbm))

    out = jax.ShapeDtypeStruct((N_PAD, W), jnp.float32)
    scratch = [
        pltpu.VMEM((EC,), jnp.int32),
        pltpu.VMEM((EC,), jnp.int32),
        pltpu.VMEM((EC,), jnp.int32),
        pltpu.VMEM((EC,), jnp.int32),
        pltpu.VMEM((EC, W), jnp.float32),
        pltpu.VMEM((EC, W), jnp.float32),
        pltpu.VMEM((ZR, W), jnp.float32),
        pltpu.VMEM_SHARED((N_PAD, W), jnp.float32),
        pltpu.SemaphoreType.DMA,
        pltpu.SemaphoreType.DMA,
        pltpu.SemaphoreType.DMA,
        pltpu.SemaphoreType.DMA,
    ]
    return functools.partial(pl.kernel, mesh=_MESH, out_type=[out, out],
                             scratch_types=scratch)(body)


def _cnt_body(dst_hbm, cnt0_hbm, cnt1_hbm, dstv, onesb, cbuf, cacc):
    c = lax.axis_index("c")
    s = lax.axis_index("s")
    w = c * 16 + s
    nc = 40

    def orow(r, _):
        for k in range(W // 16):
            onesb[r, pl.ds(k * 16, 16)] = jnp.ones((16,), jnp.float32)
        return 0
    lax.fori_loop(0, EC, orow, 0)

    def crow(r, _):
        for k in range(W // 16):
            cbuf[r, pl.ds(k * 16, 16)] = jnp.zeros((16,), jnp.float32)
        return 0
    lax.fori_loop(0, ZR, crow, 0)
    for k in range(640 // ZR):
        pltpu.sync_copy(cbuf, cacc.at[pl.ds(s * 640 + k * ZR, ZR)])
    plsc.subcore_barrier()

    def chunk(j, _):
        pltpu.sync_copy(dst_hbm.at[w * nc + j], dstv)
        pltpu.sync_copy(onesb, cacc.at[dstv], add=True)
        return 0
    lax.fori_loop(0, nc, chunk, 0)
    plsc.subcore_barrier()

    def copy_out(cnt_hbm):
        def _():
            for k in range(640 // ZR):
                r0 = s * 640 + k * ZR
                pltpu.sync_copy(cacc.at[pl.ds(r0, ZR)], cbuf)
                pltpu.sync_copy(cbuf, cnt_hbm.at[pl.ds(r0, ZR)])
        return _
    pl.when(c == 0)(copy_out(cnt0_hbm))
    pl.when(c == 1)(copy_out(cnt1_hbm))


def _make_cnt():
    out = jax.ShapeDtypeStruct((N_PAD, W), jnp.float32)
    scratch = [
        pltpu.VMEM((EC,), jnp.int32),
        pltpu.VMEM((EC, W), jnp.float32),
        pltpu.VMEM((ZR, W), jnp.float32),
        pltpu.VMEM_SHARED((N_PAD, W), jnp.float32),
    ]
    return functools.partial(
        pl.kernel, mesh=_MESH,
        out_type=[out, out],
        scratch_types=scratch)(_cnt_body)


def _decode_body(z_hbm, sidx_hbm, didx_hbm, out_hbm, sv, dv, srows, drows, sem):
    c = lax.axis_index("c")
    s = lax.axis_index("s")
    w = c * 16 + s
    pltpu.sync_copy(sidx_hbm.at[w], sv)
    pltpu.sync_copy(didx_hbm.at[w], dv)

    def chunk(j, _):
        a = pltpu.async_copy(z_hbm.at[sv.at[j]], srows, sem)
        b = pltpu.async_copy(z_hbm.at[dv.at[j]], drows, sem)
        a.wait()
        b.wait()

        def prow(r, _):
            for k in range(D_OUT // 16):
                sl = pl.ds(k * 16, 16)
                srows[r, sl] = srows[r, sl] * drows[r, sl]
            return 0
        lax.fori_loop(0, EC, prow, 0)
        pltpu.sync_copy(srows, out_hbm.at[pl.ds((w * DC + j) * EC, EC)])
        return 0
    lax.fori_loop(0, DC, chunk, 0)


def _make_decode():
    scratch = [
        pltpu.VMEM((DC, EC), jnp.int32),
        pltpu.VMEM((DC, EC), jnp.int32),
        pltpu.VMEM((EC, D_OUT), jnp.float32),
        pltpu.VMEM((EC, D_OUT), jnp.float32),
        pltpu.SemaphoreType.DMA,
    ]
    return functools.partial(
        pl.kernel, mesh=_MESH,
        out_type=jax.ShapeDtypeStruct((EL_PAD, D_OUT), jnp.float32),
        scratch_types=scratch)(_decode_body)


def _rowsum(p):
    nb, rb = 10, EL_PAD // 10

    def body(p_ref, o_ref):
        o_ref[...] = jnp.sum(p_ref[...], axis=1, keepdims=True)

    return pl.pallas_call(
        body,
        grid=(nb,),
        in_specs=[pl.BlockSpec((rb, D_OUT), lambda r: (r, 0))],
        out_specs=pl.BlockSpec((rb, 1), lambda r: (r, 0)),
        out_shape=jax.ShapeDtypeStruct((EL_PAD, 1), jnp.float32),
    )(p)


_DN = (((1,), (1,)), ((), ()))


def _matmul_split(h, Wl):
    din = h.shape[1]
    nb, rb = 10, 1000

    def body(h_ref, w_ref, o_ref):
        o_ref[...] = lax.dot_general(h_ref[...], w_ref[...], _DN,
                                     preferred_element_type=jnp.float32)

    return pl.pallas_call(
        body,
        grid=(2, nb),
        in_specs=[pl.BlockSpec((rb, din), lambda c, r: (r, 0)),
                  pl.BlockSpec((W, din), lambda c, r: (c, 0))],
        out_specs=pl.BlockSpec((rb, W), lambda c, r: (c * nb + r, 0)),
        out_shape=jax.ShapeDtypeStruct((2 * N_NODES, W), jnp.float32),
    )(h, Wl)


def _matmul_plain(h, Wl):
    din = h.shape[1]
    dout = Wl.shape[0]
    nb, rb = 10, 1000

    def body(h_ref, w_ref, o_ref):
        o_ref[...] = lax.dot_general(h_ref[...], w_ref[...], _DN,
                                     preferred_element_type=jnp.float32)

    return pl.pallas_call(
        body,
        grid=(nb,),
        in_specs=[pl.BlockSpec((rb, din), lambda r: (r, 0)),
                  pl.BlockSpec((dout, din), lambda r: (0, 0))],
        out_specs=pl.BlockSpec((rb, dout), lambda r: (r, 0)),
        out_shape=jax.ShapeDtypeStruct((N_NODES, dout), jnp.float32),
    )(h, Wl)


def _linear(h, Wmat, bl):
    dout = Wmat.shape[0]
    din = h.shape[1]
    nb, rb = 10, 1000

    def body(h_ref, w_ref, bl_ref, o_ref):
        o_ref[...] = (lax.dot_general(h_ref[...], w_ref[...], _DN,
                                      preferred_element_type=jnp.float32)
                      + bl_ref[...])

    return pl.pallas_call(
        body,
        grid=(nb,),
        in_specs=[pl.BlockSpec((rb, din), lambda r: (r, 0)),
                  pl.BlockSpec((dout, din), lambda r: (0, 0)),
                  pl.BlockSpec((1, dout), lambda r: (0, 0))],
        out_specs=pl.BlockSpec((rb, dout), lambda r: (r, 0)),
        out_shape=jax.ShapeDtypeStruct((N_NODES, dout), jnp.float32),
    )(h, Wmat, bl.reshape(1, dout))


def _finish(s_lo, s_hi, cnt_lo, cnt_hi, r, relu, mode):
    dout = r.shape[1]
    nb, rb = 10, 1000

    def body(slo, shi, clo, chi, r_ref, o_ref):
        if mode == "concat":
            sfull = jnp.concatenate([slo[...], shi[...]], axis=1)
        else:
            sfull = slo[...] + shi[...]
        cc = jnp.maximum(clo[...][:, 0:1] + chi[...][:, 0:1], 1.0)
        val = sfull / cc + r_ref[...]
        o_ref[...] = jnp.maximum(val, 0.0) if relu else val

    return pl.pallas_call(
        body,
        grid=(nb,),
        in_specs=[pl.BlockSpec((rb, W), lambda r: (r, 0)),
                  pl.BlockSpec((rb, W), lambda r: (r, 0)),
                  pl.BlockSpec((rb, W), lambda r: (r, 0)),
                  pl.BlockSpec((rb, W), lambda r: (r, 0)),
                  pl.BlockSpec((rb, dout), lambda r: (r, 0))],
        out_specs=pl.BlockSpec((rb, dout), lambda r: (r, 0)),
        out_shape=jax.ShapeDtypeStruct((N_NODES, dout), jnp.float32),
    )(s_lo, s_hi, cnt_lo, cnt_hi, r)


def kernel(x, edge_index, edge_label_index,
           Wl1, bl1, Wr1, Wl2, bl2, Wr2, Wl3, bl3, Wr3):
    src = edge_index[0]
    dst = edge_index[1]
    epad = E_PAD - E_EDGES
    src_p = jnp.concatenate([src, jnp.zeros((epad,), jnp.int32)])
    dst_p = jnp.concatenate([dst, jnp.full((epad,), DUMP_ROW, jnp.int32)])

    src_cs = jnp.concatenate([src_p, src_p + N_NODES]).reshape(2560, EC)
    dst_cs = jnp.tile(dst_p, 2).reshape(2560, EC)
    src_es = src_p.reshape(1280, EC)
    dst_es = dst_p.reshape(1280, EC)

    seg_cs = _make_seg_sum(80)
    seg_es = _make_seg_sum(40)
    cnt_k = _make_cnt()
    decode = _make_decode()

    cnt0, cnt1 = cnt_k(dst_es)

    y1 = _matmul_split(x, Wl1)
    s1a, s1b = seg_cs(y1, src_cs, dst_cs)
    r1 = _linear(x, Wr1, bl1)
    h1 = _finish(s1a, s1b, cnt0, cnt1, r1, True, "concat")

    y2 = _matmul_split(h1, Wl2)
    s2a, s2b = seg_cs(y2, src_cs, dst_cs)
    r2 = _linear(h1, Wr2, bl2)
    h2 = _finish(s2a, s2b, cnt0, cnt1, r2, True, "concat")

    y3 = _matmul_plain(h2, Wl3)
    s3a, s3b = seg_es(y3, src_es, dst_es)
    r3 = _linear(h2, Wr3, bl3)
    z = _finish(s3a, s3b, cnt0, cnt1, r3, False, "add")

    lpad = EL_PAD - EL_PAIRS
    es = jnp.concatenate([edge_label_index[0],
                          jnp.zeros((lpad,), jnp.int32)]).reshape(32, DC, EC)
    ed = jnp.concatenate([edge_label_index[1],
                          jnp.zeros((lpad,), jnp.int32)]).reshape(32, DC, EC)
    prods = decode(z, es, ed)
    dots = _rowsum(prods)
    return dots.reshape(EL_PAD)[:EL_PAIRS]


# speedup vs baseline: 1.0625x; 1.0291x over previous
import functools

import jax
import jax.numpy as jnp
from jax import lax
from jax.experimental import pallas as pl
from jax.experimental.pallas import tpu as pltpu
from jax.experimental.pallas import tpu_sc as plsc

N_NODES = 10000
N_PAD = 10240
DUMP_ROW = 10008
E_EDGES = 160000
EL_PAIRS = 20000
D_IN = 256
D_H = 256
D_OUT = 128
W = 128

EC = 128
EB = 16
E_PAD = 163840
DC = 5
EL_PAD = 32 * DC * EC
ZR = 40

_MESH = plsc.VectorSubcoreMesh(core_axis_name="c", subcore_axis_name="s")


def _make_seg_sum(nc):
    def body(y_hbm, src_hbm, dst_hbm, out0_hbm, out1_hbm,
             srcv0, srcv1, dstv0, dstv1, rows0, rows1, zbuf,
             acc, gsem0, gsem1, ssem0, ssem1):
        c = lax.axis_index("c")
        s = lax.axis_index("s")
        base = (c * 16 + s) * nc

        def zrow(r, _):
            for k in range(W // 16):
                zbuf[r, pl.ds(k * 16, 16)] = jnp.zeros((16,), jnp.float32)
            return 0
        lax.fori_loop(0, ZR, zrow, 0)
        for k in range(640 // ZR):
            pltpu.sync_copy(zbuf, acc.at[pl.ds(s * 640 + k * ZR, ZR)])
        plsc.subcore_barrier()

        pltpu.sync_copy(src_hbm.at[base], srcv0)
        pltpu.sync_copy(dst_hbm.at[base], dstv0)
        pltpu.async_copy(y_hbm.at[srcv0], rows0, gsem0)
        pltpu.sync_copy(src_hbm.at[base + 1], srcv1)
        pltpu.sync_copy(dst_hbm.at[base + 1], dstv1)
        pltpu.async_copy(y_hbm.at[srcv1], rows1, gsem1)

        def it(j2, _):
            nxt = base + 2 * j2 + 2
            pltpu.make_async_copy(y_hbm.at[srcv0], rows0, gsem0).wait()
            pltpu.async_copy(rows0, acc.at[dstv0], ssem0, add=True)
            pltpu.make_async_copy(y_hbm.at[srcv1], rows1, gsem1).wait()
            pltpu.async_copy(rows1, acc.at[dstv1], ssem1, add=True)
            pltpu.make_async_copy(rows0, acc.at[dstv0], ssem0).wait()
            pltpu.sync_copy(src_hbm.at[nxt], srcv0)
            pltpu.sync_copy(dst_hbm.at[nxt], dstv0)
            pltpu.async_copy(y_hbm.at[srcv0], rows0, gsem0)
            pltpu.make_async_copy(rows1, acc.at[dstv1], ssem1).wait()
            pltpu.sync_copy(src_hbm.at[nxt + 1], srcv1)
            pltpu.sync_copy(dst_hbm.at[nxt + 1], dstv1)
            pltpu.async_copy(y_hbm.at[srcv1], rows1, gsem1)
            return 0
        lax.fori_loop(0, nc // 2 - 1, it, 0)

        pltpu.make_async_copy(y_hbm.at[srcv0], rows0, gsem0).wait()
        pltpu.async_copy(rows0, acc.at[dstv0], ssem0, add=True)
        pltpu.make_async_copy(y_hbm.at[srcv1], rows1, gsem1).wait()
        pltpu.async_copy(rows1, acc.at[dstv1], ssem1, add=True)
        pltpu.make_async_copy(rows0, acc.at[dstv0], ssem0).wait()
        pltpu.make_async_copy(rows1, acc.at[dstv1], ssem1).wait()
        plsc.subcore_barrier()

        def copy_out(out_hbm):
            def _():
                for k in range(640 // ZR):
                    r0 = s * 640 + k * ZR
                    pltpu.sync_copy(acc.at[pl.ds(r0, ZR)], zbuf)
                    pltpu.sync_copy(zbuf, out_hbm.at[pl.ds(r0, ZR)])
            return _
        pl.when(c == 0)(copy_out(out0_hbm))
        pl.when(c == 1)(copy_out(out1_hbm))

    out = jax.ShapeDtypeStruct((N_PAD, W), jnp.float32)
    scratch = [
        pltpu.VMEM((EC,), jnp.int32),
        pltpu.VMEM((EC,), jnp.int32),
        pltpu.VMEM((EC,), jnp.int32),
        pltpu.VMEM((EC,), jnp.int32),
        pltpu.VMEM((EC, W), jnp.float32),
        pltpu.VMEM((EC, W), jnp.float32),
        pltpu.VMEM((ZR, W), jnp.float32),
        pltpu.VMEM_SHARED((N_PAD, W), jnp.float32),
        pltpu.SemaphoreType.DMA,
        pltpu.SemaphoreType.DMA,
        pltpu.SemaphoreType.DMA,
        pltpu.SemaphoreType.DMA,
    ]
    return functools.partial(pl.kernel, mesh=_MESH, out_type=[out, out],
                             scratch_types=scratch)(body)


def _cnt_body(dst_hbm, cnt0_hbm, cnt1_hbm, dstv0, dstv1, onesb, cbuf, cacc,
              ssem0, ssem1):
    c = lax.axis_index("c")
    s = lax.axis_index("s")
    w = c * 16 + s
    nc = 40

    def orow(r, _):
        for k in range(W // 16):
            onesb[r, pl.ds(k * 16, 16)] = jnp.ones((16,), jnp.float32)
        return 0
    lax.fori_loop(0, EC, orow, 0)

    def crow(r, _):
        for k in range(W // 16):
            cbuf[r, pl.ds(k * 16, 16)] = jnp.zeros((16,), jnp.float32)
        return 0
    lax.fori_loop(0, ZR, crow, 0)
    for k in range(640 // ZR):
        pltpu.sync_copy(cbuf, cacc.at[pl.ds(s * 640 + k * ZR, ZR)])
    plsc.subcore_barrier()

    base = w * nc
    pltpu.sync_copy(dst_hbm.at[base], dstv0)
    pltpu.sync_copy(dst_hbm.at[base + 1], dstv1)

    def chunk(j2, _):
        nxt = base + 2 * j2 + 2
        pltpu.async_copy(onesb, cacc.at[dstv0], ssem0, add=True)
        pltpu.async_copy(onesb, cacc.at[dstv1], ssem1, add=True)
        pltpu.make_async_copy(onesb, cacc.at[dstv0], ssem0).wait()
        pltpu.sync_copy(dst_hbm.at[nxt], dstv0)
        pltpu.make_async_copy(onesb, cacc.at[dstv1], ssem1).wait()
        pltpu.sync_copy(dst_hbm.at[nxt + 1], dstv1)
        return 0
    lax.fori_loop(0, nc // 2 - 1, chunk, 0)
    pltpu.async_copy(onesb, cacc.at[dstv0], ssem0, add=True)
    pltpu.async_copy(onesb, cacc.at[dstv1], ssem1, add=True)
    pltpu.make_async_copy(onesb, cacc.at[dstv0], ssem0).wait()
    pltpu.make_async_copy(onesb, cacc.at[dstv1], ssem1).wait()
    plsc.subcore_barrier()

    def copy_out(cnt_hbm):
        def _():
            for k in range(640 // ZR):
                r0 = s * 640 + k * ZR
                pltpu.sync_copy(cacc.at[pl.ds(r0, ZR)], cbuf)
                pltpu.sync_copy(cbuf, cnt_hbm.at[pl.ds(r0, ZR)])
        return _
    pl.when(c == 0)(copy_out(cnt0_hbm))
    pl.when(c == 1)(copy_out(cnt1_hbm))


def _make_cnt():
    out = jax.ShapeDtypeStruct((N_PAD, W), jnp.float32)
    scratch = [
        pltpu.VMEM((EC,), jnp.int32),
        pltpu.VMEM((EC,), jnp.int32),
        pltpu.VMEM((EC, W), jnp.float32),
        pltpu.VMEM((ZR, W), jnp.float32),
        pltpu.VMEM_SHARED((N_PAD, W), jnp.float32),
        pltpu.SemaphoreType.DMA,
        pltpu.SemaphoreType.DMA,
    ]
    return functools.partial(
        pl.kernel, mesh=_MESH,
        out_type=[out, out],
        scratch_types=scratch)(_cnt_body)


def _decode_body(z_hbm, sidx_hbm, didx_hbm, out_hbm, sv, dv,
                 srows0, drows0, srows1, drows1, sem0, sem1):
    c = lax.axis_index("c")
    s = lax.axis_index("s")
    w = c * 16 + s
    pltpu.sync_copy(sidx_hbm.at[w], sv)
    pltpu.sync_copy(didx_hbm.at[w], dv)
    bufs = [(srows0, drows0, sem0), (srows1, drows1, sem1)]

    def start(j):
        sr, dr, sem = bufs[j % 2]
        pltpu.async_copy(z_hbm.at[sv.at[j]], sr, sem)
        pltpu.async_copy(z_hbm.at[dv.at[j]], dr, sem)

    def finish(j):
        sr, dr, sem = bufs[j % 2]
        pltpu.make_async_copy(z_hbm.at[sv.at[j]], sr, sem).wait()
        pltpu.make_async_copy(z_hbm.at[dv.at[j]], dr, sem).wait()

        def prow(r, _):
            for k in range(D_OUT // 16):
                sl = pl.ds(k * 16, 16)
                sr[r, sl] = sr[r, sl] * dr[r, sl]
            return 0
        lax.fori_loop(0, EC, prow, 0)
        pltpu.sync_copy(sr, out_hbm.at[pl.ds((w * DC + j) * EC, EC)])

    start(0)
    start(1)
    for j in range(DC):
        finish(j)
        if j + 2 < DC:
            start(j + 2)


def _make_decode():
    scratch = [
        pltpu.VMEM((DC, EC), jnp.int32),
        pltpu.VMEM((DC, EC), jnp.int32),
        pltpu.VMEM((EC, D_OUT), jnp.float32),
        pltpu.VMEM((EC, D_OUT), jnp.float32),
        pltpu.VMEM((EC, D_OUT), jnp.float32),
        pltpu.VMEM((EC, D_OUT), jnp.float32),
        pltpu.SemaphoreType.DMA,
        pltpu.SemaphoreType.DMA,
    ]
    return functools.partial(
        pl.kernel, mesh=_MESH,
        out_type=jax.ShapeDtypeStruct((EL_PAD, D_OUT), jnp.float32),
        scratch_types=scratch)(_decode_body)


def _rowsum(p):
    nb, rb = 10, EL_PAD // 10

    def body(p_ref, o_ref):
        o_ref[...] = jnp.sum(p_ref[...], axis=1, keepdims=True)

    return pl.pallas_call(
        body,
        grid=(nb,),
        in_specs=[pl.BlockSpec((rb, D_OUT), lambda r: (r, 0))],
        out_specs=pl.BlockSpec((rb, 1), lambda r: (r, 0)),
        out_shape=jax.ShapeDtypeStruct((EL_PAD, 1), jnp.float32),
    )(p)


_DN = (((1,), (1,)), ((), ()))


def _matmul_split(h, Wl):
    din = h.shape[1]
    nb, rb = 10, 1000

    def body(h_ref, w_ref, o_ref):
        o_ref[...] = lax.dot_general(h_ref[...], w_ref[...], _DN,
                                     preferred_element_type=jnp.float32)

    return pl.pallas_call(
        body,
        grid=(2, nb),
        in_specs=[pl.BlockSpec((rb, din), lambda c, r: (r, 0)),
                  pl.BlockSpec((W, din), lambda c, r: (c, 0))],
        out_specs=pl.BlockSpec((rb, W), lambda c, r: (c * nb + r, 0)),
        out_shape=jax.ShapeDtypeStruct((2 * N_NODES, W), jnp.float32),
    )(h, Wl)


def _matmul_plain(h, Wl):
    din = h.shape[1]
    dout = Wl.shape[0]
    nb, rb = 10, 1000

    def body(h_ref, w_ref, o_ref):
        o_ref[...] = lax.dot_general(h_ref[...], w_ref[...], _DN,
                                     preferred_element_type=jnp.float32)

    return pl.pallas_call(
        body,
        grid=(nb,),
        in_specs=[pl.BlockSpec((rb, din), lambda r: (r, 0)),
                  pl.BlockSpec((dout, din), lambda r: (0, 0))],
        out_specs=pl.BlockSpec((rb, dout), lambda r: (r, 0)),
        out_shape=jax.ShapeDtypeStruct((N_NODES, dout), jnp.float32),
    )(h, Wl)


def _combine(s_lo, s_hi, cnt_lo, cnt_hi, h, Wr, bl, relu, mode):
    dout = Wr.shape[0]
    din = h.shape[1]
    nb, rb = 10, 1000

    def body(slo, shi, clo, chi, h_ref, wr_ref, bl_ref, o_ref):
        if mode == "concat":
            sfull = jnp.concatenate([slo[...], shi[...]], axis=1)
        else:
            sfull = slo[...] + shi[...]
        cc = jnp.maximum(clo[...][:, 0:1] + chi[...][:, 0:1], 1.0)
        val = (sfull / cc +
               lax.dot_general(h_ref[...], wr_ref[...], _DN,
                               preferred_element_type=jnp.float32) +
               bl_ref[...])
        o_ref[...] = jnp.maximum(val, 0.0) if relu else val

    return pl.pallas_call(
        body,
        grid=(nb,),
        in_specs=[pl.BlockSpec((rb, W), lambda r: (r, 0)),
                  pl.BlockSpec((rb, W), lambda r: (r, 0)),
                  pl.BlockSpec((rb, W), lambda r: (r, 0)),
                  pl.BlockSpec((rb, W), lambda r: (r, 0)),
                  pl.BlockSpec((rb, din), lambda r: (r, 0)),
                  pl.BlockSpec((dout, din), lambda r: (0, 0)),
                  pl.BlockSpec((1, dout), lambda r: (0, 0))],
        out_specs=pl.BlockSpec((rb, dout), lambda r: (r, 0)),
        out_shape=jax.ShapeDtypeStruct((N_NODES, dout), jnp.float32),
    )(s_lo, s_hi, cnt_lo, cnt_hi, h, Wr, bl.reshape(1, dout))


def kernel(x, edge_index, edge_label_index,
           Wl1, bl1, Wr1, Wl2, bl2, Wr2, Wl3, bl3, Wr3):
    src = edge_index[0]
    dst = edge_index[1]
    epad = E_PAD - E_EDGES
    src_p = jnp.concatenate([src, jnp.zeros((epad,), jnp.int32)])
    dst_p = jnp.concatenate([dst, jnp.full((epad,), DUMP_ROW, jnp.int32)])

    src_cs = jnp.concatenate([src_p, src_p + N_NODES]).reshape(2560, EC)
    dst_cs = jnp.tile(dst_p, 2).reshape(2560, EC)
    src_es = src_p.reshape(1280, EC)
    dst_es = dst_p.reshape(1280, EC)

    seg_cs = _make_seg_sum(80)
    seg_es = _make_seg_sum(40)
    cnt_k = _make_cnt()
    decode = _make_decode()

    cnt0, cnt1 = cnt_k(dst_es)

    y1 = _matmul_split(x, Wl1)
    s1a, s1b = seg_cs(y1, src_cs, dst_cs)
    h1 = _combine(s1a, s1b, cnt0, cnt1, x, Wr1, bl1, True, "concat")

    y2 = _matmul_split(h1, Wl2)
    s2a, s2b = seg_cs(y2, src_cs, dst_cs)
    h2 = _combine(s2a, s2b, cnt0, cnt1, h1, Wr2, bl2, True, "concat")

    y3 = _matmul_plain(h2, Wl3)
    s3a, s3b = seg_es(y3, src_es, dst_es)
    z = _combine(s3a, s3b, cnt0, cnt1, h2, Wr3, bl3, False, "add")

    lpad = EL_PAD - EL_PAIRS
    es = jnp.concatenate([edge_label_index[0],
                          jnp.zeros((lpad,), jnp.int32)]).reshape(32, DC, EC)
    ed = jnp.concatenate([edge_label_index[1],
                          jnp.zeros((lpad,), jnp.int32)]).reshape(32, DC, EC)
    prods = decode(z, es, ed)
    dots = _rowsum(prods)
    return dots.reshape(EL_PAD)[:EL_PAIRS]


# speedup vs baseline: 1.0824x; 1.0187x over previous
import functools

import jax
import jax.numpy as jnp
from jax import lax
from jax.experimental import pallas as pl
from jax.experimental.pallas import tpu as pltpu
from jax.experimental.pallas import tpu_sc as plsc

N_NODES = 10000
N_PAD = 10240
DUMP_ROW = 10008
E_EDGES = 160000
EL_PAIRS = 20000
D_IN = 256
D_H = 256
D_OUT = 128
W = 128

EC = 128
EB = 16
E_PAD = 163840
DC = 5
EL_PAD = 32 * DC * EC
ZR = 40

_MESH = plsc.VectorSubcoreMesh(core_axis_name="c", subcore_axis_name="s")


def _make_seg_sum(nc):
    def body(y_hbm, src_hbm, dst_hbm, out0_hbm, out1_hbm,
             srcv0, srcv1, dstv0, dstv1, rows0, rows1, zbuf, zbuf1,
             acc, gsem0, gsem1, ssem0, ssem1):
        c = lax.axis_index("c")
        s = lax.axis_index("s")
        base = (c * 16 + s) * nc

        def zrow(r, _):
            for k in range(W // 16):
                zbuf[r, pl.ds(k * 16, 16)] = jnp.zeros((16,), jnp.float32)
            return 0
        lax.fori_loop(0, ZR, zrow, 0)
        for k in range(640 // ZR):
            pltpu.async_copy(zbuf, acc.at[pl.ds(s * 640 + k * ZR, ZR)], ssem0)

        pltpu.sync_copy(src_hbm.at[base], srcv0)
        pltpu.sync_copy(dst_hbm.at[base], dstv0)
        pltpu.async_copy(y_hbm.at[srcv0], rows0, gsem0)
        pltpu.sync_copy(src_hbm.at[base + 1], srcv1)
        pltpu.sync_copy(dst_hbm.at[base + 1], dstv1)
        pltpu.async_copy(y_hbm.at[srcv1], rows1, gsem1)
        for k in range(640 // ZR):
            pltpu.make_async_copy(
                zbuf, acc.at[pl.ds(s * 640 + k * ZR, ZR)], ssem0).wait()
        plsc.subcore_barrier()

        def it(j2, _):
            nxt = base + 2 * j2 + 2
            pltpu.make_async_copy(y_hbm.at[srcv0], rows0, gsem0).wait()
            pltpu.async_copy(rows0, acc.at[dstv0], ssem0, add=True)
            pltpu.make_async_copy(y_hbm.at[srcv1], rows1, gsem1).wait()
            pltpu.async_copy(rows1, acc.at[dstv1], ssem1, add=True)
            pltpu.make_async_copy(rows0, acc.at[dstv0], ssem0).wait()
            pltpu.sync_copy(src_hbm.at[nxt], srcv0)
            pltpu.sync_copy(dst_hbm.at[nxt], dstv0)
            pltpu.async_copy(y_hbm.at[srcv0], rows0, gsem0)
            pltpu.make_async_copy(rows1, acc.at[dstv1], ssem1).wait()
            pltpu.sync_copy(src_hbm.at[nxt + 1], srcv1)
            pltpu.sync_copy(dst_hbm.at[nxt + 1], dstv1)
            pltpu.async_copy(y_hbm.at[srcv1], rows1, gsem1)
            return 0
        lax.fori_loop(0, nc // 2 - 1, it, 0)

        pltpu.make_async_copy(y_hbm.at[srcv0], rows0, gsem0).wait()
        pltpu.async_copy(rows0, acc.at[dstv0], ssem0, add=True)
        pltpu.make_async_copy(y_hbm.at[srcv1], rows1, gsem1).wait()
        pltpu.async_copy(rows1, acc.at[dstv1], ssem1, add=True)
        pltpu.make_async_copy(rows0, acc.at[dstv0], ssem0).wait()
        pltpu.make_async_copy(rows1, acc.at[dstv1], ssem1).wait()
        plsc.subcore_barrier()

        def copy_out(out_hbm):
            def _():
                nblk = 640 // ZR

                def sl(k):
                    return pl.ds(s * 640 + k * ZR, ZR)
                for k in range(nblk):
                    b, gsem, ssem = (zbuf, gsem0, ssem0) if k % 2 == 0 else (
                        zbuf1, gsem1, ssem1)
                    if k >= 2:
                        pltpu.make_async_copy(b, out_hbm.at[sl(k - 2)],
                                              ssem).wait()
                    pltpu.async_copy(acc.at[sl(k)], b, gsem)
                    pltpu.make_async_copy(acc.at[sl(k)], b, gsem).wait()
                    pltpu.async_copy(b, out_hbm.at[sl(k)], ssem)
                pltpu.make_async_copy(zbuf, out_hbm.at[sl(nblk - 2)],
                                      ssem0).wait()
                pltpu.make_async_copy(zbuf1, out_hbm.at[sl(nblk - 1)],
                                      ssem1).wait()
            return _
        pl.when(c == 0)(copy_out(out0_hbm))
        pl.when(c == 1)(copy_out(out1_hbm))

    out = jax.ShapeDtypeStruct((N_PAD, W), jnp.float32)
    scratch = [
        pltpu.VMEM((EC,), jnp.int32),
        pltpu.VMEM((EC,), jnp.int32),
        pltpu.VMEM((EC,), jnp.int32),
        pltpu.VMEM((EC,), jnp.int32),
        pltpu.VMEM((EC, W), jnp.float32),
        pltpu.VMEM((EC, W), jnp.float32),
        pltpu.VMEM((ZR, W), jnp.float32),
        pltpu.VMEM((ZR, W), jnp.float32),
        pltpu.VMEM_SHARED((N_PAD, W), jnp.float32),
        pltpu.SemaphoreType.DMA,
        pltpu.SemaphoreType.DMA,
        pltpu.SemaphoreType.DMA,
        pltpu.SemaphoreType.DMA,
    ]
    return functools.partial(pl.kernel, mesh=_MESH, out_type=[out, out],
                             scratch_types=scratch)(body)


def _cnt_body(dst_hbm, cnt0_hbm, cnt1_hbm, dstv0, dstv1, onesb, cbuf, cacc,
              ssem0, ssem1):
    c = lax.axis_index("c")
    s = lax.axis_index("s")
    w = c * 16 + s
    nc = 40

    def orow(r, _):
        for k in range(W // 16):
            onesb[r, pl.ds(k * 16, 16)] = jnp.ones((16,), jnp.float32)
        return 0
    lax.fori_loop(0, EC, orow, 0)

    def crow(r, _):
        for k in range(W // 16):
            cbuf[r, pl.ds(k * 16, 16)] = jnp.zeros((16,), jnp.float32)
        return 0
    lax.fori_loop(0, ZR, crow, 0)
    for k in range(640 // ZR):
        pltpu.sync_copy(cbuf, cacc.at[pl.ds(s * 640 + k * ZR, ZR)])
    plsc.subcore_barrier()

    base = w * nc
    pltpu.sync_copy(dst_hbm.at[base], dstv0)
    pltpu.sync_copy(dst_hbm.at[base + 1], dstv1)

    def chunk(j2, _):
        nxt = base + 2 * j2 + 2
        pltpu.async_copy(onesb, cacc.at[dstv0], ssem0, add=True)
        pltpu.async_copy(onesb, cacc.at[dstv1], ssem1, add=True)
        pltpu.make_async_copy(onesb, cacc.at[dstv0], ssem0).wait()
        pltpu.sync_copy(dst_hbm.at[nxt], dstv0)
        pltpu.make_async_copy(onesb, cacc.at[dstv1], ssem1).wait()
        pltpu.sync_copy(dst_hbm.at[nxt + 1], dstv1)
        return 0
    lax.fori_loop(0, nc // 2 - 1, chunk, 0)
    pltpu.async_copy(onesb, cacc.at[dstv0], ssem0, add=True)
    pltpu.async_copy(onesb, cacc.at[dstv1], ssem1, add=True)
    pltpu.make_async_copy(onesb, cacc.at[dstv0], ssem0).wait()
    pltpu.make_async_copy(onesb, cacc.at[dstv1], ssem1).wait()
    plsc.subcore_barrier()

    def copy_out(cnt_hbm):
        def _():
            for k in range(640 // ZR):
                r0 = s * 640 + k * ZR
                pltpu.sync_copy(cacc.at[pl.ds(r0, ZR)], cbuf)
                pltpu.sync_copy(cbuf, cnt_hbm.at[pl.ds(r0, ZR)])
        return _
    pl.when(c == 0)(copy_out(cnt0_hbm))
    pl.when(c == 1)(copy_out(cnt1_hbm))


def _make_cnt():
    out = jax.ShapeDtypeStruct((N_PAD, W), jnp.float32)
    scratch = [
        pltpu.VMEM((EC,), jnp.int32),
        pltpu.VMEM((EC,), jnp.int32),
        pltpu.VMEM((EC, W), jnp.float32),
        pltpu.VMEM((ZR, W), jnp.float32),
        pltpu.VMEM_SHARED((N_PAD, W), jnp.float32),
        pltpu.SemaphoreType.DMA,
        pltpu.SemaphoreType.DMA,
    ]
    return functools.partial(
        pl.kernel, mesh=_MESH,
        out_type=[out, out],
        scratch_types=scratch)(_cnt_body)


def _decode_body(z_hbm, sidx_hbm, didx_hbm, out_hbm, sv, dv,
                 srows0, drows0, srows1, drows1, sem0, sem1):
    c = lax.axis_index("c")
    s = lax.axis_index("s")
    w = c * 16 + s
    pltpu.sync_copy(sidx_hbm.at[w], sv)
    pltpu.sync_copy(didx_hbm.at[w], dv)
    bufs = [(srows0, drows0, sem0), (srows1, drows1, sem1)]

    def start(j):
        sr, dr, sem = bufs[j % 2]
        pltpu.async_copy(z_hbm.at[sv.at[j]], sr, sem)
        pltpu.async_copy(z_hbm.at[dv.at[j]], dr, sem)

    def finish(j):
        sr, dr, sem = bufs[j % 2]
        pltpu.make_async_copy(z_hbm.at[sv.at[j]], sr, sem).wait()
        pltpu.make_async_copy(z_hbm.at[dv.at[j]], dr, sem).wait()

        def prow(r, _):
            for k in range(D_OUT // 16):
                sl = pl.ds(k * 16, 16)
                sr[r, sl] = sr[r, sl] * dr[r, sl]
            return 0
        lax.fori_loop(0, EC, prow, 0)
        pltpu.sync_copy(sr, out_hbm.at[pl.ds((w * DC + j) * EC, EC)])

    start(0)
    start(1)
    for j in range(DC):
        finish(j)
        if j + 2 < DC:
            start(j + 2)


def _make_decode():
    scratch = [
        pltpu.VMEM((DC, EC), jnp.int32),
        pltpu.VMEM((DC, EC), jnp.int32),
        pltpu.VMEM((EC, D_OUT), jnp.float32),
        pltpu.VMEM((EC, D_OUT), jnp.float32),
        pltpu.VMEM((EC, D_OUT), jnp.float32),
        pltpu.VMEM((EC, D_OUT), jnp.float32),
        pltpu.SemaphoreType.DMA,
        pltpu.SemaphoreType.DMA,
    ]
    return functools.partial(
        pl.kernel, mesh=_MESH,
        out_type=jax.ShapeDtypeStruct((EL_PAD, D_OUT), jnp.float32),
        scratch_types=scratch)(_decode_body)


def _rowsum(p):
    nb, rb = 10, EL_PAD // 10

    def body(p_ref, o_ref):
        o_ref[...] = jnp.sum(p_ref[...], axis=1, keepdims=True)

    return pl.pallas_call(
        body,
        grid=(nb,),
        in_specs=[pl.BlockSpec((rb, D_OUT), lambda r: (r, 0))],
        out_specs=pl.BlockSpec((rb, 1), lambda r: (r, 0)),
        out_shape=jax.ShapeDtypeStruct((EL_PAD, 1), jnp.float32),
    )(p)


_DN = (((1,), (1,)), ((), ()))


def _matmul_split(h, Wl):
    din = h.shape[1]
    nb, rb = 10, 1000

    def body(h_ref, w_ref, o_ref):
        o_ref[...] = lax.dot_general(h_ref[...], w_ref[...], _DN,
                                     preferred_element_type=jnp.float32)

    return pl.pallas_call(
        body,
        grid=(2, nb),
        in_specs=[pl.BlockSpec((rb, din), lambda c, r: (r, 0)),
                  pl.BlockSpec((W, din), lambda c, r: (c, 0))],
        out_specs=pl.BlockSpec((rb, W), lambda c, r: (c * nb + r, 0)),
        out_shape=jax.ShapeDtypeStruct((2 * N_NODES, W), jnp.float32),
    )(h, Wl)


def _matmul_plain(h, Wl):
    din = h.shape[1]
    dout = Wl.shape[0]
    nb, rb = 10, 1000

    def body(h_ref, w_ref, o_ref):
        o_ref[...] = lax.dot_general(h_ref[...], w_ref[...], _DN,
                                     preferred_element_type=jnp.float32)

    return pl.pallas_call(
        body,
        grid=(nb,),
        in_specs=[pl.BlockSpec((rb, din), lambda r: (r, 0)),
                  pl.BlockSpec((dout, din), lambda r: (0, 0))],
        out_specs=pl.BlockSpec((rb, dout), lambda r: (r, 0)),
        out_shape=jax.ShapeDtypeStruct((N_NODES, dout), jnp.float32),
    )(h, Wl)


def _combine(s_lo, s_hi, cnt_lo, cnt_hi, h, Wr, bl, relu, mode):
    dout = Wr.shape[0]
    din = h.shape[1]
    nb, rb = 10, 1000

    def body(slo, shi, clo, chi, h_ref, wr_ref, bl_ref, o_ref):
        if mode == "concat":
            sfull = jnp.concatenate([slo[...], shi[...]], axis=1)
        else:
            sfull = slo[...] + shi[...]
        cc = jnp.maximum(clo[...][:, 0:1] + chi[...][:, 0:1], 1.0)
        val = (sfull / cc +
               lax.dot_general(h_ref[...], wr_ref[...], _DN,
                               preferred_element_type=jnp.float32) +
               bl_ref[...])
        o_ref[...] = jnp.maximum(val, 0.0) if relu else val

    return pl.pallas_call(
        body,
        grid=(nb,),
        in_specs=[pl.BlockSpec((rb, W), lambda r: (r, 0)),
                  pl.BlockSpec((rb, W), lambda r: (r, 0)),
                  pl.BlockSpec((rb, W), lambda r: (r, 0)),
                  pl.BlockSpec((rb, W), lambda r: (r, 0)),
                  pl.BlockSpec((rb, din), lambda r: (r, 0)),
                  pl.BlockSpec((dout, din), lambda r: (0, 0)),
                  pl.BlockSpec((1, dout), lambda r: (0, 0))],
        out_specs=pl.BlockSpec((rb, dout), lambda r: (r, 0)),
        out_shape=jax.ShapeDtypeStruct((N_NODES, dout), jnp.float32),
    )(s_lo, s_hi, cnt_lo, cnt_hi, h, Wr, bl.reshape(1, dout))


def kernel(x, edge_index, edge_label_index,
           Wl1, bl1, Wr1, Wl2, bl2, Wr2, Wl3, bl3, Wr3):
    src = edge_index[0]
    dst = edge_index[1]
    epad = E_PAD - E_EDGES
    src_p = jnp.concatenate([src, jnp.zeros((epad,), jnp.int32)])
    dst_p = jnp.concatenate([dst, jnp.full((epad,), DUMP_ROW, jnp.int32)])

    src_cs = jnp.concatenate([src_p, src_p + N_NODES]).reshape(2560, EC)
    dst_cs = jnp.tile(dst_p, 2).reshape(2560, EC)
    src_es = src_p.reshape(1280, EC)
    dst_es = dst_p.reshape(1280, EC)

    seg_cs = _make_seg_sum(80)
    seg_es = _make_seg_sum(40)
    cnt_k = _make_cnt()
    decode = _make_decode()

    cnt0, cnt1 = cnt_k(dst_es)

    y1 = _matmul_split(x, Wl1)
    s1a, s1b = seg_cs(y1, src_cs, dst_cs)
    h1 = _combine(s1a, s1b, cnt0, cnt1, x, Wr1, bl1, True, "concat")

    y2 = _matmul_split(h1, Wl2)
    s2a, s2b = seg_cs(y2, src_cs, dst_cs)
    h2 = _combine(s2a, s2b, cnt0, cnt1, h1, Wr2, bl2, True, "concat")

    y3 = _matmul_plain(h2, Wl3)
    s3a, s3b = seg_es(y3, src_es, dst_es)
    z = _combine(s3a, s3b, cnt0, cnt1, h2, Wr3, bl3, False, "add")

    lpad = EL_PAD - EL_PAIRS
    es = jnp.concatenate([edge_label_index[0],
                          jnp.zeros((lpad,), jnp.int32)]).reshape(32, DC, EC)
    ed = jnp.concatenate([edge_label_index[1],
                          jnp.zeros((lpad,), jnp.int32)]).reshape(32, DC, EC)
    prods = decode(z, es, ed)
    dots = _rowsum(prods)
    return dots.reshape(EL_PAD)[:EL_PAIRS]
